# baseline (speedup 1.0000x reference)
import jax
import jax.numpy as jnp
from jax import lax
from jax.experimental import pallas as pl
from jax.experimental.pallas import tpu as pltpu
from jax.experimental.pallas import tpu_sc as plsc

_B = 16384
_D = 32
_NC = 2
_NS = 16
_NW = _NC * _NS
_BW = _B // _NW
_GROUPS = _BW // 16


def _dot_sigmoid_kernel(x0_hbm, x1_hbm, u_hbm, m_hbm, wb_hbm, out_hbm,
                        idx_u, idx_m, u_rows, m_rows, out_v, wb_v,
                        sem_u, sem_m):
    wid = lax.axis_index("s") * _NC + lax.axis_index("c")
    base = wid * _BW

    pltpu.sync_copy(x0_hbm.at[pl.ds(base, _BW)], idx_u)
    pltpu.sync_copy(x1_hbm.at[pl.ds(base, _BW)], idx_m)
    pltpu.sync_copy(wb_hbm, wb_v)

    cu = pltpu.async_copy(u_hbm.at[idx_u], u_rows, sem_u)
    cm = pltpu.async_copy(m_hbm.at[idx_m], m_rows, sem_m)
    cu.wait()
    cm.wait()

    lanes = lax.iota(jnp.int32, 16)
    w = wb_v[pl.ds(0, 16)]
    b = wb_v[pl.ds(16, 16)]
    one = jnp.ones((16,), jnp.float32)

    def group_body(g, carry):
        rows = g * 16 + lanes
        acc0 = jnp.zeros((16,), jnp.float32)
        acc1 = jnp.zeros((16,), jnp.float32)
        acc2 = jnp.zeros((16,), jnp.float32)
        acc3 = jnp.zeros((16,), jnp.float32)
        accs = [acc0, acc1, acc2, acc3]
        for d in range(_D):
            col = jnp.full((16,), d, jnp.int32)
            uv = plsc.load_gather(u_rows, [rows, col])
            mv = plsc.load_gather(m_rows, [rows, col])
            accs[d % 4] = accs[d % 4] + uv * mv
        z = (accs[0] + accs[1]) + (accs[2] + accs[3])
        t = z * w + b
        r = one / (one + jnp.exp(-t))
        out_v[pl.ds(g * 16, 16)] = r
        return carry

    lax.fori_loop(0, _GROUPS, group_body, 0)

    pltpu.sync_copy(out_v, out_hbm.at[pl.ds(base, _BW)])


def kernel(x, u_table, m_table, fc_w, fc_b):
    x = x.astype(jnp.int32)
    x0 = x[0]
    x1 = x[1]
    wb = jnp.concatenate([
        jnp.broadcast_to(fc_w.reshape(-1)[:1], (16,)),
        jnp.broadcast_to(fc_b.reshape(-1)[:1], (16,)),
    ]).astype(jnp.float32)

    mesh = plsc.VectorSubcoreMesh(core_axis_name="c", subcore_axis_name="s")
    run = pl.kernel(
        _dot_sigmoid_kernel,
        out_type=jax.ShapeDtypeStruct((_B,), jnp.float32),
        mesh=mesh,
        compiler_params=pltpu.CompilerParams(
            needs_layout_passes=False, use_tc_tiling_on_sc=False
        ),
        scratch_types=[
            pltpu.VMEM((_BW,), jnp.int32),
            pltpu.VMEM((_BW,), jnp.int32),
            pltpu.VMEM((_BW, _D), jnp.float32),
            pltpu.VMEM((_BW, _D), jnp.float32),
            pltpu.VMEM((_BW,), jnp.float32),
            pltpu.VMEM((32,), jnp.float32),
            pltpu.SemaphoreType.DMA,
            pltpu.SemaphoreType.DMA,
        ],
    )
    out = run(x0, x1, u_table, m_table, wb)
    return out.reshape(_B, 1)
